# baseline (speedup 1.0000x reference)
import jax
import jax.numpy as jnp
from jax.experimental import pallas as pl
from jax.experimental.pallas import tpu as pltpu

N = 2048
H1 = 8
F1 = 32
D1 = H1 * F1
F2 = 32
BI = 256
PW = 128


def _leaky(x):
    return jnp.where(x >= 0, x, 0.2 * x)


def _body(x_ref, adj_ref, w1_ref, al_ref, ar_ref, w2_ref, a2l_ref, a2r_ref,
          out_ref,
          gbx_s, el_s, er1_s, ermax_s, bmat_s, dmat_s, gsum_s,
          g2b_s, el2_s, er2_s, gsum2_s, em2_s, bvec2_s, dvec2_s):
    p = pl.program_id(0)
    i = pl.program_id(1)
    nb = pl.num_programs(1)

    @pl.when(p == 0)
    def _proj():
        sl = pl.ds(i * BI, BI)
        g = jnp.dot(x_ref[...], w1_ref[...], preferred_element_type=jnp.float32)
        gbf = g.astype(jnp.bfloat16)
        one_col = jnp.ones((BI, 1), jnp.bfloat16)
        for h in range(H1):
            gbx_s[sl, PW * h:PW * h + F1] = gbf[:, h * F1:(h + 1) * F1]
            gbx_s[sl, PW * h + F1:PW * h + F1 + 1] = one_col
        el_s[sl, :] = jnp.dot(g, al_ref[...], preferred_element_type=jnp.float32)
        er1_s[sl, :] = jnp.dot(g, ar_ref[...], preferred_element_type=jnp.float32)
        colsum = jnp.sum(g, axis=0, keepdims=True)

        @pl.when(i == 0)
        def _init():
            gsum_s[...] = colsum

        @pl.when(i != 0)
        def _acc():
            gsum_s[...] += colsum

        @pl.when(i == nb - 1)
        def _finish():
            ert = er1_s[...].T
            ermax = jnp.max(ert, axis=1, keepdims=True)
            ermax_s[...] = ermax
            bmat_s[...] = jnp.exp(ert - ermax).astype(jnp.bfloat16)
            dmat_s[...] = jnp.exp(0.2 * (ert - ermax)).astype(jnp.bfloat16)
            g2b_s[:, F2:] = jnp.ones((N, 1), jnp.bfloat16)

    @pl.when(p == 1)
    def _attn1():
        maskb = adj_ref[...].astype(jnp.bfloat16)
        el = el_s[pl.ds(i * BI, BI), :]
        ermax = ermax_s[...]
        gmean = gsum_s[...] * (1.0 / N)
        outs = []
        for h in range(H1):
            x = el[:, h:h + 1] + ermax[h:h + 1, :]
            bound = _leaky(x)
            a = jnp.exp(x - bound).astype(jnp.bfloat16)
            c = jnp.exp(0.2 * x - bound).astype(jnp.bfloat16)
            w = jnp.maximum(a * bmat_s[h:h + 1, :],
                            c * dmat_s[h:h + 1, :]) * maskb
            r = jnp.dot(w, gbx_s[:, PW * h:PW * h + F1 + 1],
                        preferred_element_type=jnp.float32)
            num = r[:, :F1]
            denom = r[:, F1:F1 + 1]
            outs.append(jnp.where(denom > 0, num / denom,
                                  gmean[:, h * F1:(h + 1) * F1]))
        hcat = jnp.concatenate(outs, axis=1)
        hact = jnp.where(hcat > 0, hcat, jnp.exp(hcat) - 1.0)
        g2 = jnp.dot(hact, w2_ref[...], preferred_element_type=jnp.float32)
        sl = pl.ds(i * BI, BI)
        g2b_s[sl, :F2] = g2.astype(jnp.bfloat16)
        el2_s[sl, :] = jnp.dot(g2, a2l_ref[...], preferred_element_type=jnp.float32)
        er2_s[sl, :] = jnp.dot(g2, a2r_ref[...], preferred_element_type=jnp.float32)
        colsum2 = jnp.sum(g2, axis=0, keepdims=True)

        @pl.when(i == 0)
        def _init():
            gsum2_s[...] = colsum2

        @pl.when(i != 0)
        def _acc():
            gsum2_s[...] += colsum2

    @pl.when((p == 2) & (i == 0))
    def _prep2():
        ert = er2_s[...].T
        em = jnp.max(ert, axis=1, keepdims=True)
        em2_s[...] = em
        bvec2_s[...] = jnp.exp(ert - em).astype(jnp.bfloat16)
        dvec2_s[...] = jnp.exp(0.2 * (ert - em)).astype(jnp.bfloat16)

    @pl.when(p == 2)
    def _attn2():
        maskb = adj_ref[...].astype(jnp.bfloat16)
        el = el2_s[pl.ds(i * BI, BI), :]
        em = em2_s[...]
        x = el + em
        bound = _leaky(x)
        a = jnp.exp(x - bound).astype(jnp.bfloat16)
        c = jnp.exp(0.2 * x - bound).astype(jnp.bfloat16)
        w = jnp.maximum(a * bvec2_s[...], c * dvec2_s[...]) * maskb
        r = jnp.dot(w, g2b_s[...], preferred_element_type=jnp.float32)
        denom = r[:, F2:F2 + 1]
        gmean = gsum2_s[...] * (1.0 / N)
        out_ref[...] = jnp.where(denom > 0, r[:, :F2] / denom, gmean)


def kernel(x, adj_mat, W1, a1_l, a1_r, W2, a2_l, a2_r):
    adj = adj_mat.reshape(N, N).astype(jnp.int8)
    eye = jnp.eye(H1, dtype=jnp.float32)
    al1 = (eye[:, None, :] * a1_l[None, :, None]).reshape(D1, H1)
    ar1 = (eye[:, None, :] * a1_r[None, :, None]).reshape(D1, H1)
    a2l = a2_l.reshape(F2, 1)
    a2r = a2_r.reshape(F2, 1)

    nb = N // BI
    out = pl.pallas_call(
        _body,
        grid=(3, nb),
        in_specs=[
            pl.BlockSpec((BI, x.shape[1]),
                         lambda p, i: (jnp.where(p == 0, i, nb - 1), 0)),
            pl.BlockSpec((BI, N), lambda p, i: (jnp.where(p == 0, 0, i), 0)),
            pl.BlockSpec(W1.shape, lambda p, i: (0, 0)),
            pl.BlockSpec((D1, H1), lambda p, i: (0, 0)),
            pl.BlockSpec((D1, H1), lambda p, i: (0, 0)),
            pl.BlockSpec(W2.shape, lambda p, i: (0, 0)),
            pl.BlockSpec((F2, 1), lambda p, i: (0, 0)),
            pl.BlockSpec((F2, 1), lambda p, i: (0, 0)),
        ],
        out_specs=pl.BlockSpec((BI, F2),
                               lambda p, i: (jnp.where(p == 2, i, 0), 0)),
        out_shape=jax.ShapeDtypeStruct((N, F2), jnp.float32),
        scratch_shapes=[
            pltpu.VMEM((N, PW * H1), jnp.bfloat16),
            pltpu.VMEM((N, H1), jnp.float32),
            pltpu.VMEM((N, H1), jnp.float32),
            pltpu.VMEM((H1, 1), jnp.float32),
            pltpu.VMEM((H1, N), jnp.bfloat16),
            pltpu.VMEM((H1, N), jnp.bfloat16),
            pltpu.VMEM((1, D1), jnp.float32),
            pltpu.VMEM((N, F2 + 1), jnp.bfloat16),
            pltpu.VMEM((N, 1), jnp.float32),
            pltpu.VMEM((N, 1), jnp.float32),
            pltpu.VMEM((1, F2), jnp.float32),
            pltpu.VMEM((1, 1), jnp.float32),
            pltpu.VMEM((1, N), jnp.bfloat16),
            pltpu.VMEM((1, N), jnp.bfloat16),
        ],
    )(x, adj, W1, al1, ar1, W2, a2l, a2r)
    return out


# speedup vs baseline: 1.0660x; 1.0660x over previous
import jax
import jax.numpy as jnp
from jax.experimental import pallas as pl
from jax.experimental.pallas import tpu as pltpu

N = 2048
H1 = 8
F1 = 32
D1 = H1 * F1
F2 = 32
BI = 256


def _leaky(x):
    return jnp.where(x >= 0, x, 0.2 * x)


def _body(x_ref, adj_ref, w1_ref, al_ref, ar_ref, w2_ref, a2l_ref, a2r_ref,
          out_ref,
          gb_s, el_s, ermax_s, bmat_s, dmat_s, gsum_s,
          g2b_s, el2_s, er2_s, gsum2_s):
    p = pl.program_id(0)
    i = pl.program_id(1)

    @pl.when((p == 0) & (i == 0))
    def _proj():
        g = jnp.dot(x_ref[...], w1_ref[...], preferred_element_type=jnp.float32)
        gb_s[...] = g.astype(jnp.bfloat16)
        el_s[...] = jnp.dot(g, al_ref[...], preferred_element_type=jnp.float32)
        er = jnp.dot(g, ar_ref[...], preferred_element_type=jnp.float32)
        ert = er.T
        ermax = jnp.max(ert, axis=1, keepdims=True)
        ermax_s[...] = ermax
        bmat_s[...] = jnp.exp(ert - ermax).astype(jnp.bfloat16)
        dmat_s[...] = jnp.exp(0.2 * (ert - ermax)).astype(jnp.bfloat16)
        gsum_s[...] = jnp.sum(g, axis=0, keepdims=True)
        g2b_s[:, F2:] = jnp.ones((N, 1), jnp.bfloat16)

    @pl.when(p == 0)
    def _attn1():
        maskb = adj_ref[...].astype(jnp.bfloat16)
        gb = gb_s[...]
        el = el_s[pl.ds(i * BI, BI), :]
        ermax = ermax_s[...]
        gmean = gsum_s[...] * (1.0 / N)
        ones = jnp.ones((N, 1), jnp.bfloat16)
        outs = []
        for h in range(H1):
            x = el[:, h:h + 1] + ermax[h:h + 1, :]
            bound = _leaky(x)
            a = jnp.exp(x - bound).astype(jnp.bfloat16)
            c = jnp.exp(0.2 * x - bound).astype(jnp.bfloat16)
            w = jnp.maximum(a * bmat_s[h:h + 1, :],
                            c * dmat_s[h:h + 1, :]) * maskb
            gbh = jnp.concatenate([gb[:, h * F1:(h + 1) * F1], ones], axis=1)
            r = jnp.dot(w, gbh, preferred_element_type=jnp.float32)
            num = r[:, :F1]
            denom = r[:, F1:F1 + 1]
            outs.append(jnp.where(denom > 0, num / denom,
                                  gmean[:, h * F1:(h + 1) * F1]))
        hcat = jnp.concatenate(outs, axis=1)
        hact = jnp.where(hcat > 0, hcat, jnp.exp(hcat) - 1.0)
        g2 = jnp.dot(hact, w2_ref[...], preferred_element_type=jnp.float32)
        g2b_s[pl.ds(i * BI, BI), :F2] = g2.astype(jnp.bfloat16)
        el2_s[pl.ds(i * BI, BI), :] = jnp.dot(
            g2, a2l_ref[...], preferred_element_type=jnp.float32)
        er2_s[pl.ds(i * BI, BI), :] = jnp.dot(
            g2, a2r_ref[...], preferred_element_type=jnp.float32)
        colsum2 = jnp.sum(g2, axis=0, keepdims=True)

        @pl.when(i == 0)
        def _init():
            gsum2_s[...] = colsum2

        @pl.when(i != 0)
        def _acc():
            gsum2_s[...] += colsum2

    @pl.when(p == 1)
    def _attn2():
        maskb = adj_ref[...].astype(jnp.bfloat16)
        el = el2_s[pl.ds(i * BI, BI), :]
        ert = er2_s[...].T
        em = jnp.max(ert, axis=1, keepdims=True)
        bvec = jnp.exp(ert - em).astype(jnp.bfloat16)
        dvec = jnp.exp(0.2 * (ert - em)).astype(jnp.bfloat16)
        x = el + em
        bound = _leaky(x)
        a = jnp.exp(x - bound).astype(jnp.bfloat16)
        c = jnp.exp(0.2 * x - bound).astype(jnp.bfloat16)
        w = jnp.maximum(a * bvec, c * dvec) * maskb
        r = jnp.dot(w, g2b_s[...], preferred_element_type=jnp.float32)
        denom = r[:, F2:F2 + 1]
        gmean = gsum2_s[...] * (1.0 / N)
        out_ref[...] = jnp.where(denom > 0, r[:, :F2] / denom, gmean)


def kernel(x, adj_mat, W1, a1_l, a1_r, W2, a2_l, a2_r):
    adj = adj_mat.reshape(N, N).astype(jnp.int8)
    eye = jnp.eye(H1, dtype=jnp.float32)
    al1 = (eye[:, None, :] * a1_l[None, :, None]).reshape(D1, H1)
    ar1 = (eye[:, None, :] * a1_r[None, :, None]).reshape(D1, H1)
    a2l = a2_l.reshape(F2, 1)
    a2r = a2_r.reshape(F2, 1)

    nb = N // BI
    out = pl.pallas_call(
        _body,
        grid=(2, nb),
        in_specs=[
            pl.BlockSpec((N, x.shape[1]), lambda p, i: (0, 0)),
            pl.BlockSpec((BI, N), lambda p, i: (i, 0)),
            pl.BlockSpec(W1.shape, lambda p, i: (0, 0)),
            pl.BlockSpec((D1, H1), lambda p, i: (0, 0)),
            pl.BlockSpec((D1, H1), lambda p, i: (0, 0)),
            pl.BlockSpec(W2.shape, lambda p, i: (0, 0)),
            pl.BlockSpec((F2, 1), lambda p, i: (0, 0)),
            pl.BlockSpec((F2, 1), lambda p, i: (0, 0)),
        ],
        out_specs=pl.BlockSpec((BI, F2), lambda p, i: (i, 0)),
        out_shape=jax.ShapeDtypeStruct((N, F2), jnp.float32),
        scratch_shapes=[
            pltpu.VMEM((N, D1), jnp.bfloat16),
            pltpu.VMEM((N, H1), jnp.float32),
            pltpu.VMEM((H1, 1), jnp.float32),
            pltpu.VMEM((H1, N), jnp.bfloat16),
            pltpu.VMEM((H1, N), jnp.bfloat16),
            pltpu.VMEM((1, D1), jnp.float32),
            pltpu.VMEM((N, F2 + 1), jnp.bfloat16),
            pltpu.VMEM((N, 1), jnp.float32),
            pltpu.VMEM((N, 1), jnp.float32),
            pltpu.VMEM((1, F2), jnp.float32),
        ],
    )(x, adj, W1, al1, ar1, W2, a2l, a2r)
    return out
